# baseline (speedup 1.0000x reference)
import functools

import jax
import jax.numpy as jnp
from jax import lax
from jax.experimental import pallas as pl
from jax.experimental.pallas import tpu as pltpu
from jax.experimental.pallas import tpu_sc as plsc

N = 100000
E = 1600000
DH = 16
NS = 16
NP = 100096
CHUNK = 128
NCH = 3
GE = CHUNK * NCH
PE = 2 * GE
PAIRS = 134
EPT = PE * PAIRS
EP = NS * EPT
RPT = NP // NS
RCH = 272
NRC = RPT // RCH

_mesh = plsc.VectorSubcoreMesh(core_axis_name="c", subcore_axis_name="s")


@functools.partial(
    pl.kernel,
    mesh=_mesh,
    compiler_params=pltpu.CompilerParams(use_tc_tiling_on_sc=False),
    out_type=[
        jax.ShapeDtypeStruct((2 * NP, DH), jnp.float32),
        jax.ShapeDtypeStruct((2 * NP, DH), jnp.float32),
    ],
    scratch_types=[
        pltpu.VMEM((2, PE), jnp.int32),
        pltpu.VMEM((2, 2 * NCH, CHUNK), jnp.int32),
        pltpu.VMEM((2, PE), jnp.float32),
        pltpu.VMEM((2, PE, DH), jnp.float32),
        pltpu.VMEM_SHARED((NP, DH), jnp.float32),
        pltpu.SemaphoreType.DMA,
        pltpu.SemaphoreType.DMA,
        pltpu.SemaphoreType.DMA,
        pltpu.SemaphoreType.DMA,
    ],
)
def _sc_body(x0s, colsh, row2d, w, x1, out,
             colbuf, rowbuf, wbuf, gbuf, accum, gsem0, gsem1, ssemA, ssemB):
    cid = lax.axis_index("c")
    sid = lax.axis_index("s")
    coff = cid * NP
    ceoff = cid * EP
    e0t = sid * EPT
    r0t = sid * RPT

    abuf = gbuf.at[0, pl.ds(0, RCH), :]
    bbuf = gbuf.at[0, pl.ds(RCH, RCH), :]
    cbuf = gbuf.at[1, pl.ds(0, RCH), :]

    def fill_abuf_zero():
        def zrow(i, _):
            gbuf[0, i, :] = jnp.zeros((DH,), jnp.float32)
            return 0
        lax.fori_loop(0, RCH, zrow, 0)

    def zero_accum():
        fill_abuf_zero()

        def zchunk(i, _):
            pltpu.sync_copy(
                abuf, accum.at[pl.ds(pl.multiple_of(r0t + i * RCH, 8), RCH)])
            return 0
        lax.fori_loop(0, NRC, zchunk, 0)

    def spmm_layer(src_hbm):
        def load_idx(e0, a):
            pltpu.sync_copy(colsh.at[pl.ds(ceoff + e0, PE)], colbuf.at[a])
            pltpu.sync_copy(w.at[pl.ds(e0, PE)], wbuf.at[a])
            c0 = pl.multiple_of(e0 // CHUNK, 2 * NCH)
            pltpu.sync_copy(row2d.at[pl.ds(c0, 2 * NCH)], rowbuf.at[a])

        def fire_gathers(a):
            return [pltpu.async_copy(
                src_hbm.at[colbuf.at[a, pl.ds(i * CHUNK, CHUNK)]],
                gbuf.at[a, pl.ds(i * CHUNK, CHUNK), :],
                gsem0 if i < NCH else gsem1) for i in range(2 * NCH)]

        def mul_half(a, base):
            def mul16(b, _):
                b16 = pl.multiple_of(base + b * 16, 16)
                wv = wbuf[a, pl.ds(b16, 16)]
                for k in range(16):
                    gbuf[a, b16 + k, :] = gbuf[a, b16 + k, :] * wv[k]
                return 0
            lax.fori_loop(0, GE // 16, mul16, 0)

        def fire_scatters(a, half, ssem):
            return [pltpu.async_copy(
                gbuf.at[a, pl.ds((half * NCH + i) * CHUNK, CHUNK), :],
                accum.at[rowbuf.at[a, half * NCH + i]],
                ssem, add=True) for i in range(NCH)]

        def process(a, gath, ssem):
            for cp in gath[:NCH]:
                cp.wait()
            mul_half(a, 0)
            sc0 = fire_scatters(a, 0, ssem)
            for cp in gath[NCH:]:
                cp.wait()
            mul_half(a, GE)
            sc1 = fire_scatters(a, 1, ssem)
            return sc0 + sc1

        def body(j, _):
            eA = pl.multiple_of(e0t + (2 * j) * PE, PE)
            eB = pl.multiple_of(eA + PE, PE)
            load_idx(eA, 0)
            gathA = fire_gathers(0)
            load_idx(eB, 1)
            scA = process(0, gathA, ssemA)
            gathB = fire_gathers(1)
            scB = process(1, gathB, ssemB)
            for cp in scA + scB:
                cp.wait()
            return 0
        lax.fori_loop(0, PAIRS // 2, body, 0)

    zero_accum()
    plsc.subcore_barrier()
    spmm_layer(x0s)
    plsc.subcore_barrier()

    fill_abuf_zero()

    def wb(i, _):
        r0 = pl.multiple_of(r0t + i * RCH, 8)
        pltpu.sync_copy(accum.at[pl.ds(r0, RCH)], bbuf)
        pltpu.sync_copy(bbuf, x1.at[pl.ds(coff + r0, RCH)])
        pltpu.sync_copy(abuf, accum.at[pl.ds(r0, RCH)])
        return 0
    lax.fori_loop(0, NRC, wb, 0)
    plsc.subcore_barrier()

    spmm_layer(x1)
    plsc.subcore_barrier()

    def comb(i, _):
        r0 = pl.multiple_of(r0t + i * RCH, 8)
        pltpu.sync_copy(x0s.at[pl.ds(coff + r0, RCH)], abuf)
        pltpu.sync_copy(x1.at[pl.ds(coff + r0, RCH)], bbuf)
        pltpu.sync_copy(accum.at[pl.ds(r0, RCH)], cbuf)

        def crow(r, _):
            gbuf[1, r, :] = \
                (gbuf[0, r, :] + gbuf[0, RCH + r, :] + gbuf[1, r, :]) \
                * jnp.float32(1.0 / 3.0)
            return 0
        lax.fori_loop(0, RCH, crow, 0)
        pltpu.sync_copy(cbuf, out.at[pl.ds(coff + r0, RCH)])
        return 0
    lax.fori_loop(0, NRC, comb, 0)


def kernel(edge_index, edge_weight, embedding_weight):
    row = edge_index[0].astype(jnp.int32)
    col = edge_index[1].astype(jnp.int32)
    pad = EP - E
    spread = jnp.arange(pad, dtype=jnp.int32) % N
    row = jnp.concatenate([row, spread])
    col = jnp.concatenate([col, spread])
    w = jnp.concatenate([edge_weight, jnp.zeros((pad,), jnp.float32)])
    row2d = row.reshape(EP // CHUNK, CHUNK)
    colsh = jnp.concatenate([col, col + NP])
    zrows = jnp.zeros((NP - N, DH), jnp.float32)
    x0s = jnp.concatenate(
        [embedding_weight[:, :DH], zrows, embedding_weight[:, DH:], zrows],
        axis=0)
    _x1, outs = _sc_body(x0s, colsh, row2d, w)
    return jnp.concatenate([outs[:N], outs[NP:NP + N]], axis=1)


# speedup vs baseline: 1.0889x; 1.0889x over previous
import functools

import jax
import jax.numpy as jnp
from jax import lax
from jax.experimental import pallas as pl
from jax.experimental.pallas import tpu as pltpu
from jax.experimental.pallas import tpu_sc as plsc

N = 100000
E = 1600000
DH = 16
NS = 16
NP = 100096
EP = 1646592
CHUNK = 128
NCH = 6
GE = CHUNK * NCH
PE = 2 * GE
EPT = EP // NS
NG = EPT // GE
RPT = NP // NS
RCH = 272
NRC = RPT // RCH

_mesh = plsc.VectorSubcoreMesh(core_axis_name="c", subcore_axis_name="s")


@functools.partial(
    pl.kernel,
    mesh=_mesh,
    compiler_params=pltpu.CompilerParams(use_tc_tiling_on_sc=False),
    out_type=[
        jax.ShapeDtypeStruct((2 * NP, DH), jnp.float32),
        jax.ShapeDtypeStruct((2 * NP, DH), jnp.float32),
    ],
    scratch_types=[
        pltpu.VMEM((PE,), jnp.int32),
        pltpu.VMEM((2 * NCH, CHUNK), jnp.int32),
        pltpu.VMEM((PE,), jnp.float32),
        pltpu.VMEM((PE, DH), jnp.float32),
        pltpu.VMEM_SHARED((NP, DH), jnp.float32),
        pltpu.SemaphoreType.DMA,
        pltpu.SemaphoreType.DMA,
        pltpu.SemaphoreType.DMA,
    ],
)
def _sc_body(x0s, colsh, row2d, w, x1, out,
             colbuf, rowbuf, wbuf, gbuf, accum, gsem0, gsem1, ssem):
    cid = lax.axis_index("c")
    sid = lax.axis_index("s")
    coff = cid * NP
    ceoff = cid * EP
    e0t = sid * EPT
    r0t = sid * RPT

    abuf = gbuf.at[pl.ds(0, RCH), :]
    bbuf = gbuf.at[pl.ds(RCH, RCH), :]
    cbuf = gbuf.at[pl.ds(2 * RCH, RCH), :]

    def fill_abuf_zero():
        def zrow(i, _):
            gbuf[i, :] = jnp.zeros((DH,), jnp.float32)
            return 0
        lax.fori_loop(0, RCH, zrow, 0)

    def zero_accum():
        fill_abuf_zero()

        def zchunk(i, _):
            pltpu.sync_copy(
                abuf, accum.at[pl.ds(pl.multiple_of(r0t + i * RCH, 8), RCH)])
            return 0
        lax.fori_loop(0, NRC, zchunk, 0)

    def spmm_layer(src_hbm):
        def mul_half(base):
            def mul16(b, _):
                b16 = pl.multiple_of(base + b * 16, 16)
                wv = wbuf[pl.ds(b16, 16)]
                for k in range(16):
                    gbuf[b16 + k, :] = gbuf[b16 + k, :] * wv[k]
                return 0
            lax.fori_loop(0, GE // 16, mul16, 0)

        def fire_scatters(half):
            return [pltpu.async_copy(
                gbuf.at[pl.ds((half * NCH + i) * CHUNK, CHUNK), :],
                accum.at[rowbuf.at[half * NCH + i]],
                ssem, add=True) for i in range(NCH)]

        def pair(p, _):
            e0 = pl.multiple_of(e0t + p * PE, PE)
            pltpu.sync_copy(colsh.at[pl.ds(ceoff + e0, PE)], colbuf)
            pltpu.sync_copy(w.at[pl.ds(e0, PE)], wbuf)
            c0 = pl.multiple_of(e0 // CHUNK, 2 * NCH)
            pltpu.sync_copy(row2d.at[pl.ds(c0, 2 * NCH)], rowbuf)

            gath = [pltpu.async_copy(
                src_hbm.at[colbuf.at[pl.ds(i * CHUNK, CHUNK)]],
                gbuf.at[pl.ds(i * CHUNK, CHUNK), :],
                gsem0 if i < NCH else gsem1) for i in range(2 * NCH)]

            for cp in gath[:NCH]:
                cp.wait()
            mul_half(0)
            sc0 = fire_scatters(0)
            for cp in gath[NCH:]:
                cp.wait()
            mul_half(GE)
            sc1 = fire_scatters(1)
            for cp in sc0 + sc1:
                cp.wait()
            return 0
        lax.fori_loop(0, NG // 2, pair, 0)

    zero_accum()
    plsc.subcore_barrier()
    spmm_layer(x0s)
    plsc.subcore_barrier()

    fill_abuf_zero()

    def wb(i, _):
        r0 = pl.multiple_of(r0t + i * RCH, 8)
        pltpu.sync_copy(accum.at[pl.ds(r0, RCH)], bbuf)
        pltpu.sync_copy(bbuf, x1.at[pl.ds(coff + r0, RCH)])
        pltpu.sync_copy(abuf, accum.at[pl.ds(r0, RCH)])
        return 0
    lax.fori_loop(0, NRC, wb, 0)
    plsc.subcore_barrier()

    spmm_layer(x1)
    plsc.subcore_barrier()

    def comb(i, _):
        r0 = pl.multiple_of(r0t + i * RCH, 8)
        pltpu.sync_copy(x0s.at[pl.ds(coff + r0, RCH)], abuf)
        pltpu.sync_copy(x1.at[pl.ds(coff + r0, RCH)], bbuf)
        pltpu.sync_copy(accum.at[pl.ds(r0, RCH)], cbuf)

        def crow(r, _):
            gbuf[2 * RCH + r, :] = \
                (gbuf[r, :] + gbuf[RCH + r, :] + gbuf[2 * RCH + r, :]) \
                * jnp.float32(1.0 / 3.0)
            return 0
        lax.fori_loop(0, RCH, crow, 0)
        pltpu.sync_copy(cbuf, out.at[pl.ds(coff + r0, RCH)])
        return 0
    lax.fori_loop(0, NRC, comb, 0)


def kernel(edge_index, edge_weight, embedding_weight):
    row = edge_index[0].astype(jnp.int32)
    col = edge_index[1].astype(jnp.int32)
    pad = EP - E
    spread = jnp.arange(pad, dtype=jnp.int32) % N
    row = jnp.concatenate([row, spread])
    col = jnp.concatenate([col, spread])
    w = jnp.concatenate([edge_weight, jnp.zeros((pad,), jnp.float32)])
    row2d = row.reshape(EP // CHUNK, CHUNK)
    colsh = jnp.concatenate([col, col + NP])
    zrows = jnp.zeros((NP - N, DH), jnp.float32)
    x0s = jnp.concatenate(
        [embedding_weight[:, :DH], zrows, embedding_weight[:, DH:], zrows],
        axis=0)
    _x1, outs = _sc_body(x0s, colsh, row2d, w)
    return jnp.concatenate([outs[:N], outs[NP:NP + N]], axis=1)
